# baseline (speedup 1.0000x reference)
import functools

import numpy as np
import jax
import jax.numpy as jnp
from jax import lax
from jax.experimental import pallas as pl
from jax.experimental.pallas import tpu as pltpu
from jax.experimental.pallas import tpu_sc as plsc

N = 10000
E = 320000
D = 128
DROP = int(0.7 * E)

NC, NS, L = 2, 16, 16
NW = NC * NS
RPW = 80
R2D = NW * RPW
E_PAD = R2D * 128
PADE = E_PAD - E
RPS = R2D // NS
NPADH = 10240
NPT = NPADH // NS
NACC = 10240
NROWS_T = NACC // NS

_PAD_IDX = np.arange(PADE, dtype=np.int32) % N


def _rsqrt_vec(x):
    y = 1.0 / x
    for _ in range(26):
        y = y * (1.5 - 0.5 * x * y * y)
    return y


def _prep_body(src_hbm, dst_hbm, w_hbm, keep_hbm, wp_hbm,
               src_v, dst_v, w_v, keep_v, wp_v,
               onorm_v, inorm_v, tmp_v, zeros_v, ones_v,
               hout_s, hin_s):
    c = lax.axis_index("c")
    s = lax.axis_index("s")
    wid = s * NC + c

    pltpu.sync_copy(src_hbm.at[pl.ds(s * RPS, RPS)], src_v)
    pltpu.sync_copy(dst_hbm.at[pl.ds(s * RPS, RPS)], dst_v)

    zv = jnp.zeros((L,), jnp.float32)
    ov = jnp.ones((L,), jnp.float32)

    def _fill_zero(i, _):
        zeros_v[pl.ds(i * L, L)] = zv
        return 0
    lax.fori_loop(0, NPT // L, _fill_zero, 0)
    for k in range(128 // L):
        ones_v[pl.ds(k * L, L)] = ov

    pltpu.sync_copy(zeros_v, hout_s.at[pl.ds(s * NPT, NPT)])
    pltpu.sync_copy(zeros_v, hin_s.at[pl.ds(s * NPT, NPT)])
    plsc.subcore_barrier()

    def _hist(j, _):
        pltpu.sync_copy(ones_v, hout_s.at[src_v.at[j]], add=True)
        pltpu.sync_copy(ones_v, hin_s.at[dst_v.at[j]], add=True)
        return 0
    lax.fori_loop(0, RPS, _hist, 0)
    plsc.subcore_barrier()

    iota = lax.iota(jnp.int32, L)
    for hist in (hout_s, hin_s):
        pltpu.sync_copy(hist.at[pl.ds(s * NPT, NPT)], tmp_v)

        def _norm(i, _):
            hv = tmp_v[pl.ds(i * L, L)]
            nvec = s * NPT + i * L + iota
            padc = jnp.where(nvec < PADE, 1.0, 0.0).astype(jnp.float32)
            deg = jnp.maximum(hv - padc, 1.0)
            tmp_v[pl.ds(i * L, L)] = _rsqrt_vec(deg)
            return 0
        lax.fori_loop(0, NPT // L, _norm, 0)
        pltpu.sync_copy(tmp_v, hist.at[pl.ds(s * NPT, NPT)])
    plsc.subcore_barrier()

    pltpu.sync_copy(hout_s, onorm_v)
    pltpu.sync_copy(hin_s, inorm_v)

    r0 = wid * RPW
    loc = c * RPW
    pltpu.sync_copy(w_hbm.at[pl.ds(r0, RPW)], w_v)
    pltpu.sync_copy(keep_hbm.at[pl.ds(r0, RPW)], keep_v)

    def _wp(j, _):
        for k in range(128 // L):
            sl = pl.ds(k * L, L)
            sidx = src_v[loc + j, sl]
            didx = dst_v[loc + j, sl]
            on = plsc.load_gather(onorm_v, [sidx])
            inr = plsc.load_gather(inorm_v, [didx])
            wp_v[j, sl] = w_v[j, sl] * keep_v[j, sl] * on * inr
        return 0
    lax.fori_loop(0, RPW, _wp, 0)
    pltpu.sync_copy(wp_v, wp_hbm.at[pl.ds(r0, RPW)])


_prep = pl.kernel(
    _prep_body,
    out_type=jax.ShapeDtypeStruct((R2D, 128), jnp.float32),
    compiler_params=pltpu.CompilerParams(needs_layout_passes=False, use_tc_tiling_on_sc=False),
    mesh=plsc.VectorSubcoreMesh(core_axis_name="c", subcore_axis_name="s"),
    scratch_types=[
        pltpu.VMEM((RPS, 128), jnp.int32),
        pltpu.VMEM((RPS, 128), jnp.int32),
        pltpu.VMEM((RPW, 128), jnp.float32),
        pltpu.VMEM((RPW, 128), jnp.float32),
        pltpu.VMEM((RPW, 128), jnp.float32),
        pltpu.VMEM((NPADH,), jnp.float32),
        pltpu.VMEM((NPADH,), jnp.float32),
        pltpu.VMEM((NPT,), jnp.float32),
        pltpu.VMEM((NPT,), jnp.float32),
        pltpu.VMEM((128,), jnp.float32),
        pltpu.VMEM_SHARED((NPADH,), jnp.float32),
        pltpu.VMEM_SHARED((NPADH,), jnp.float32),
    ],
)


DH = D // NC
NCHT = RPS
RB = 4
PF = 2


def _spmm_body(tbl0, tbl1, pk_hbm, wp_hbm, p_hbm,
               pk_v, wp_v, sidx_v, didx_v,
               rb0, rb1, rb2, rb3, gsem, ssem, acc_s):
    c = lax.axis_index("c")
    s = lax.axis_index("s")
    r0 = s * RPS
    bufs = (rb0, rb1, rb2, rb3)

    pltpu.sync_copy(pk_hbm.at[pl.ds(r0, RPS)], pk_v)
    pltpu.sync_copy(wp_hbm.at[pl.ds(r0, RPS)], wp_v)

    zv = jnp.zeros((L,), jnp.float32)

    def _zrow(i, _):
        for k in range(DH // L):
            rb0[i, pl.ds(k * L, L)] = zv
        return 0
    lax.fori_loop(0, 128, _zrow, 0)
    for q in range(NROWS_T // 128):
        pltpu.sync_copy(rb0, acc_s.at[pl.ds(s * NROWS_T + q * 128, 128)])
    plsc.subcore_barrier()

    def _gather_start(j, buf, slot):
        for k in range(128 // L):
            sl = pl.ds(k * L, L)
            pkv = pk_v[j, sl]
            sidx_v[slot, sl] = pkv & 0x3FFF
            didx_v[slot, sl] = pkv >> 14

        @pl.when(c == 0)
        def _():
            pltpu.async_copy(tbl0.at[sidx_v.at[slot]], buf, gsem)

        @pl.when(c == 1)
        def _():
            pltpu.async_copy(tbl1.at[sidx_v.at[slot]], buf, gsem)

    def _gather_wait(buf):
        pltpu.make_async_copy(tbl0.at[pl.ds(0, 128)], buf, gsem).wait()

    def _scatter_start(buf, slot):
        pltpu.async_copy(buf, acc_s.at[didx_v.at[slot]], ssem, add=True)

    def _scatter_drain(buf):
        pltpu.make_async_copy(tbl0.at[pl.ds(0, 128)], buf, ssem).wait()

    def _scale(j, buf):
        def _grp(g, _):
            wv = wp_v[j, pl.ds(g * L, L)]
            for rr in range(L):
                wsc = wv[rr]
                for k in range(DH // L):
                    sl = pl.ds(k * L, L)
                    buf[g * L + rr, sl] = buf[g * L + rr, sl] * wsc
            return 0
        lax.fori_loop(0, 128 // L, _grp, 0)

    _gather_start(0, bufs[0], 0)
    _gather_start(1, bufs[1], 1)

    def _sub(j, b):
        buf = bufs[b]
        _gather_wait(buf)
        _scale(j, buf)
        _scatter_start(buf, b)

    for b in range(RB):
        j0 = b
        _sub(j0, b)
        if j0 >= PF:
            _scatter_drain(bufs[0])
        _gather_start(j0 + PF, bufs[(j0 + PF) % RB], (j0 + PF) % RB)

    def _outer(go, _):
        for b in range(RB):
            j = go * RB + b
            _sub(j, b)
            _scatter_drain(bufs[0])
            _gather_start(j + PF, bufs[(b + PF) % RB], (b + PF) % RB)
        return 0
    lax.fori_loop(1, NCHT // RB - 1, _outer, 0)

    for b in range(RB):
        j = NCHT - RB + b
        _sub(j, b)
        _scatter_drain(bufs[0])
        if j + PF < NCHT:
            _gather_start(j + PF, bufs[(b + PF) % RB], (b + PF) % RB)

    for _ in range(PF):
        _scatter_drain(bufs[0])
    plsc.subcore_barrier()

    pltpu.sync_copy(acc_s.at[pl.ds(s * NROWS_T, NROWS_T)],
                    p_hbm.at[c, pl.ds(s * NROWS_T, NROWS_T)])


_spmm = pl.kernel(
    _spmm_body,
    out_type=jax.ShapeDtypeStruct((NC, NACC, DH), jnp.float32),
    compiler_params=pltpu.CompilerParams(needs_layout_passes=False, use_tc_tiling_on_sc=False),
    mesh=plsc.VectorSubcoreMesh(core_axis_name="c", subcore_axis_name="s"),
    scratch_types=[
        pltpu.VMEM((RPS, 128), jnp.int32),
        pltpu.VMEM((RPS, 128), jnp.float32),
        pltpu.VMEM((RB, 128), jnp.int32),
        pltpu.VMEM((RB, 128), jnp.int32),
        pltpu.VMEM((128, DH), jnp.float32),
        pltpu.VMEM((128, DH), jnp.float32),
        pltpu.VMEM((128, DH), jnp.float32),
        pltpu.VMEM((128, DH), jnp.float32),
        pltpu.SemaphoreType.DMA,
        pltpu.SemaphoreType.DMA,
        pltpu.VMEM_SHARED((NACC, DH), jnp.float32),
    ],
)


def _final_body(f_ref, p1_ref, p2_ref, o_ref):
    h1 = jnp.concatenate([p1_ref[0], p1_ref[1]], axis=-1)
    h2 = jnp.concatenate([p2_ref[0], p2_ref[1]], axis=-1)
    o_ref[...] = (f_ref[...] + h1 + h2) * (1.0 / 3.0)


_final = pl.pallas_call(
    _final_body,
    grid=(10,),
    in_specs=[pl.BlockSpec((1000, D), lambda i: (i, 0)),
              pl.BlockSpec((2, 1000, DH), lambda i: (0, i, 0)),
              pl.BlockSpec((2, 1000, DH), lambda i: (0, i, 0))],
    out_specs=pl.BlockSpec((1000, D), lambda i: (i, 0)),
    out_shape=jax.ShapeDtypeStruct((N, D), jnp.float32),
)


def kernel(feature, edge_index, w):
    ridx = jax.random.randint(jax.random.key(1), (DROP,), 0, E)
    keep = jnp.ones((E,), jnp.float32).at[ridx].set(0.0)
    keep2d = jnp.concatenate(
        [keep, jnp.zeros((PADE,), jnp.float32)]).reshape(R2D, 128)

    src = edge_index[0]
    dst = edge_index[1]
    pad_idx = jnp.asarray(_PAD_IDX)
    srcp = jnp.concatenate([src, pad_idx]).reshape(R2D, 128)
    dstp = jnp.concatenate([dst, pad_idx]).reshape(R2D, 128)
    wpad = jnp.concatenate([w, jnp.zeros((PADE,), w.dtype)]).reshape(R2D, 128)

    wprime = _prep(srcp, dstp, wpad, keep2d)
    packed = jnp.bitwise_or(srcp, dstp << 14)
    fpad = jnp.concatenate(
        [feature, jnp.zeros((NACC - N, D), feature.dtype)], axis=0)
    p1 = _spmm(fpad[:, :DH], fpad[:, DH:], packed, wprime)
    p2 = _spmm(p1[0], p1[1], packed, wprime)
    return _final(feature, p1, p2)


# speedup vs baseline: 4.3966x; 4.3966x over previous
import jax
import jax.numpy as jnp
from jax import lax
from jax.experimental import pallas as pl
from jax.experimental.pallas import tpu as pltpu
from jax.experimental.pallas import tpu_sc as plsc

N = 10000
E = 320000
D = 128
DROP = int(0.7 * E)

NC, NS, L = 2, 16, 16
NW = NC * NS
RPW = 80
R2D = NW * RPW
E_PAD = R2D * 128
PADE = E_PAD - E
RPS = R2D // NS
NPT = 640
NACC = 10240
NROWS_T = NACC // NS
W2DR = E // 128
RIDX_ROWS = 1792
RIDX_PT = RIDX_ROWS // NS
CPW = 5248
KCH = CPW // 128
CBUF = CPW + L
FSUM = NW * 320

_SC_PARAMS = pltpu.CompilerParams(
    needs_layout_passes=False, use_tc_tiling_on_sc=False)
_MESH = plsc.VectorSubcoreMesh(core_axis_name="c", subcore_axis_name="s")


def _rsqrt_vec(x):
    y = 1.0 / x
    for _ in range(26):
        y = y * (1.5 - 0.5 * x * y * y)
    return y


def _prep_body(src_hbm, dst_hbm, w_hbm, ridx_hbm, pk_out, wp_out,
               src_v, dst_v, w_v, keep_v, ridx_v, pkc_v, wpc_v,
               onorm_v, inorm_v, tmp_v, ones_v,
               keep_s, hout_s, hin_s):
    c = lax.axis_index("c")
    s = lax.axis_index("s")
    wid = s * NC + c
    iota = lax.iota(jnp.int32, L)

    pltpu.sync_copy(src_hbm.at[pl.ds(s * RPS, RPS)], src_v)
    pltpu.sync_copy(dst_hbm.at[pl.ds(s * RPS, RPS)], dst_v)

    zv = jnp.zeros((L,), jnp.float32)
    ov = jnp.ones((L,), jnp.float32)
    for k in range(128 // L):
        ones_v[pl.ds(k * L, L)] = ov

    def _fill_zero(i, _):
        tmp_v[pl.ds(i * L, L)] = zv
        return 0
    lax.fori_loop(0, NPT // L, _fill_zero, 0)

    def _fill_one(i, _):
        keep_v[pl.ds(i * L, L)] = ov
        return 0
    lax.fori_loop(0, NACC // L, _fill_one, 0)

    pltpu.sync_copy(tmp_v, hout_s.at[pl.ds(s * NPT, NPT)])
    pltpu.sync_copy(tmp_v, hin_s.at[pl.ds(s * NPT, NPT)])
    kbase = s * (E_PAD // NS)
    pltpu.sync_copy(keep_v, keep_s.at[pl.ds(kbase, NACC)])
    pltpu.sync_copy(keep_v, keep_s.at[pl.ds(kbase + NACC, NACC)])
    plsc.subcore_barrier()

    pltpu.sync_copy(ridx_hbm.at[pl.ds(s * RIDX_PT, RIDX_PT)], ridx_v)

    def _drop(j, _):
        pltpu.sync_copy(tmp_v.at[pl.ds(0, 128)], keep_s.at[ridx_v.at[j]])
        return 0
    lax.fori_loop(0, RIDX_PT, _drop, 0)

    def _hist(j, _):
        pltpu.sync_copy(ones_v, hout_s.at[src_v.at[j]], add=True)
        pltpu.sync_copy(ones_v, hin_s.at[dst_v.at[j]], add=True)
        return 0
    lax.fori_loop(0, RPS, _hist, 0)
    plsc.subcore_barrier()

    for hist in (hout_s, hin_s):
        pltpu.sync_copy(hist.at[pl.ds(s * NPT, NPT)], tmp_v)

        def _norm(i, _):
            hv = tmp_v[pl.ds(i * L, L)]
            nvec = s * NPT + i * L + iota
            padc = jnp.where(nvec < PADE, 1.0, 0.0).astype(jnp.float32)
            deg = jnp.maximum(hv - padc, 1.0)
            tmp_v[pl.ds(i * L, L)] = _rsqrt_vec(deg)
            return 0
        lax.fori_loop(0, NPT // L, _norm, 0)
        pltpu.sync_copy(tmp_v, hist.at[pl.ds(s * NPT, NPT)])
    plsc.subcore_barrier()

    pltpu.sync_copy(hout_s, onorm_v)
    pltpu.sync_copy(hin_s, inorm_v)
    pltpu.sync_copy(keep_s.at[pl.ds(wid * (RPW * 128), RPW * 128)], keep_v)

    start = jnp.minimum(wid * RPW, W2DR - RPW)
    woff = wid * RPW - start
    pltpu.sync_copy(w_hbm.at[pl.ds(start, RPW)], w_v)

    def _prefill(i, _):
        v = (wid * CPW + i * L + iota) % N
        pkc_v[pl.ds(i * L, L)] = v | (v << 14)
        wpc_v[pl.ds(i * L, L)] = zv
        return 0
    lax.fori_loop(0, CBUF // L, _prefill, 0)

    loc = c * RPW

    def _wp(j, off):
        jr = jnp.minimum(woff + j, RPW - 1)
        for k in range(128 // L):
            sl = pl.ds(k * L, L)
            sidx = src_v[loc + j, sl]
            didx = dst_v[loc + j, sl]
            on = plsc.load_gather(onorm_v, [sidx])
            inr = plsc.load_gather(inorm_v, [didx])
            kv = keep_v[pl.ds(j * 128 + k * L, L)]
            wv = w_v[jr, sl]
            eid = (wid * RPW + j) * 128 + k * L + iota
            m = (kv != 0.0) & (eid < E)
            wp = wv * kv * on * inr
            pk = sidx | (didx << 14)
            plsc.store_compressed(pkc_v.at[pl.ds(off, L)], pk, mask=m)
            plsc.store_compressed(wpc_v.at[pl.ds(off, L)], wp, mask=m)
            off = off + plsc.all_reduce_population_count(m)[0]
        return off
    lax.fori_loop(0, RPW, _wp, jnp.int32(0))

    pltpu.sync_copy(pkc_v.at[pl.ds(0, CPW)], pk_out.at[pl.ds(wid * CPW, CPW)])
    pltpu.sync_copy(wpc_v.at[pl.ds(0, CPW)], wp_out.at[pl.ds(wid * CPW, CPW)])


_prep = pl.kernel(
    _prep_body,
    out_type=(jax.ShapeDtypeStruct((NW * CPW,), jnp.int32),
              jax.ShapeDtypeStruct((NW * CPW,), jnp.float32)),
    compiler_params=_SC_PARAMS,
    mesh=_MESH,
    scratch_types=[
        pltpu.VMEM((RPS, 128), jnp.int32),
        pltpu.VMEM((RPS, 128), jnp.int32),
        pltpu.VMEM((RPW, 128), jnp.float32),
        pltpu.VMEM((NACC,), jnp.float32),
        pltpu.VMEM((RIDX_PT, 128), jnp.int32),
        pltpu.VMEM((CBUF,), jnp.int32),
        pltpu.VMEM((CBUF,), jnp.float32),
        pltpu.VMEM((NACC,), jnp.float32),
        pltpu.VMEM((NACC,), jnp.float32),
        pltpu.VMEM((NPT,), jnp.float32),
        pltpu.VMEM((128,), jnp.float32),
        pltpu.VMEM_SHARED((E_PAD,), jnp.float32),
        pltpu.VMEM_SHARED((NACC,), jnp.float32),
        pltpu.VMEM_SHARED((NACC,), jnp.float32),
    ],
)


def _spmm_body(tbl, pk_hbm, wp_hbm, p_hbm,
               pk_v, wp_v, sidx_v, didx_v, rb0, rb1, gsem, ssem, acc_s):
    c = lax.axis_index("c")
    s = lax.axis_index("s")
    wid = s * NC + c
    bufs = (rb0, rb1)

    pltpu.sync_copy(pk_hbm.at[pl.ds(wid * CPW, CPW)], pk_v)
    pltpu.sync_copy(wp_hbm.at[pl.ds(wid * CPW, CPW)], wp_v)

    zv = jnp.zeros((L,), jnp.float32)

    def _zrow(i, _):
        for k in range(D // L):
            rb0[i, pl.ds(k * L, L)] = zv
        return 0
    lax.fori_loop(0, 128, _zrow, 0)
    for q in range(NROWS_T // 128):
        pltpu.sync_copy(rb0, acc_s.at[pl.ds(s * NROWS_T + q * 128, 128)])
    plsc.subcore_barrier()

    def _gather_start(j, b):
        for k in range(128 // L):
            sl = pl.ds(k * L, L)
            pkv = pk_v[pl.ds(j * 128 + k * L, L)]
            sidx_v[b, sl] = pkv & 0x3FFF
            didx_v[b, sl] = pkv >> 14
        pltpu.async_copy(tbl.at[sidx_v.at[b]], bufs[b], gsem)

    def _gather_wait(b):
        pltpu.make_async_copy(tbl.at[pl.ds(0, 128)], bufs[b], gsem).wait()

    def _scatter_start(b):
        pltpu.async_copy(bufs[b], acc_s.at[didx_v.at[b]], ssem, add=True)

    def _scatter_drain():
        pltpu.make_async_copy(tbl.at[pl.ds(0, 128)], rb0, ssem).wait()

    def _scale(j, b):
        buf = bufs[b]

        def _grp(g, _):
            wv = wp_v[pl.ds(j * 128 + g * L, L)]
            for rr in range(L):
                wsc = wv[rr]
                for k in range(D // L):
                    sl = pl.ds(k * L, L)
                    buf[g * L + rr, sl] = buf[g * L + rr, sl] * wsc
            return 0
        lax.fori_loop(0, 128 // L, _grp, 0)

    def _sub(j, b, drain, gnext):
        _gather_wait(b)
        _scale(j, b)
        _scatter_start(b)
        if drain:
            _scatter_drain()
        if gnext:
            _gather_start(j + 1, 1 - b)

    _gather_start(0, 0)
    _sub(0, 0, False, True)

    def _outer(jo, _):
        j = 1 + 2 * jo
        _sub(j, 1, True, True)
        _sub(j + 1, 0, True, True)
        return 0
    lax.fori_loop(0, (KCH - 3) // 2, _outer, 0)

    _sub(KCH - 2, 1, True, True)
    _sub(KCH - 1, 0, True, False)
    _scatter_drain()
    plsc.subcore_barrier()

    pltpu.sync_copy(acc_s.at[pl.ds(s * NROWS_T, NROWS_T)],
                    p_hbm.at[c, pl.ds(s * NROWS_T, NROWS_T)])


_spmm = pl.kernel(
    _spmm_body,
    out_type=jax.ShapeDtypeStruct((NC, NACC, D), jnp.float32),
    compiler_params=_SC_PARAMS,
    mesh=_MESH,
    scratch_types=[
        pltpu.VMEM((CPW,), jnp.int32),
        pltpu.VMEM((CPW,), jnp.float32),
        pltpu.VMEM((2, 128), jnp.int32),
        pltpu.VMEM((2, 128), jnp.int32),
        pltpu.VMEM((128, D), jnp.float32),
        pltpu.VMEM((128, D), jnp.float32),
        pltpu.SemaphoreType.DMA,
        pltpu.SemaphoreType.DMA,
        pltpu.VMEM_SHARED((NACC, D), jnp.float32),
    ],
)


def _sum2_body(p_hbm, h_hbm, a_v, b_v):
    c = lax.axis_index("c")
    s = lax.axis_index("s")
    wid = s * NC + c
    base = wid * (NACC // NW)

    def _chunk(q, _):
        r = base + q * 64
        pltpu.sync_copy(p_hbm.at[0, pl.ds(r, 64)], a_v)
        pltpu.sync_copy(p_hbm.at[1, pl.ds(r, 64)], b_v)

        def _row(i, _):
            for k in range(D // L):
                sl = pl.ds(k * L, L)
                a_v[i, sl] = a_v[i, sl] + b_v[i, sl]
            return 0
        lax.fori_loop(0, 64, _row, 0)
        pltpu.sync_copy(a_v, h_hbm.at[pl.ds(r, 64)])
        return 0
    lax.fori_loop(0, (NACC // NW) // 64, _chunk, 0)


_sum2 = pl.kernel(
    _sum2_body,
    out_type=jax.ShapeDtypeStruct((NACC, D), jnp.float32),
    compiler_params=_SC_PARAMS,
    mesh=_MESH,
    scratch_types=[
        pltpu.VMEM((64, D), jnp.float32),
        pltpu.VMEM((64, D), jnp.float32),
    ],
)


def _final_body(f_hbm, h1_hbm, p2_hbm, o_hbm, a_v, b_v):
    c = lax.axis_index("c")
    s = lax.axis_index("s")
    wid = s * NC + c
    base = jnp.minimum(wid * 320, N - 320)

    def _chunk(q, _):
        r = base + q * 64
        pltpu.sync_copy(h1_hbm.at[pl.ds(r, 64)], a_v)
        pltpu.sync_copy(p2_hbm.at[0, pl.ds(r, 64)], b_v)

        def _row1(i, _):
            for k in range(D // L):
                sl = pl.ds(k * L, L)
                a_v[i, sl] = a_v[i, sl] + b_v[i, sl]
            return 0
        lax.fori_loop(0, 64, _row1, 0)
        pltpu.sync_copy(p2_hbm.at[1, pl.ds(r, 64)], b_v)

        def _row2(i, _):
            for k in range(D // L):
                sl = pl.ds(k * L, L)
                a_v[i, sl] = a_v[i, sl] + b_v[i, sl]
            return 0
        lax.fori_loop(0, 64, _row2, 0)
        pltpu.sync_copy(f_hbm.at[pl.ds(r, 64)], b_v)

        def _row3(i, _):
            for k in range(D // L):
                sl = pl.ds(k * L, L)
                a_v[i, sl] = (a_v[i, sl] + b_v[i, sl]) * (1.0 / 3.0)
            return 0
        lax.fori_loop(0, 64, _row3, 0)
        pltpu.sync_copy(a_v, o_hbm.at[pl.ds(r, 64)])
        return 0
    lax.fori_loop(0, 5, _chunk, 0)


_final = pl.kernel(
    _final_body,
    out_type=jax.ShapeDtypeStruct((N, D), jnp.float32),
    compiler_params=_SC_PARAMS,
    mesh=_MESH,
    scratch_types=[
        pltpu.VMEM((64, D), jnp.float32),
        pltpu.VMEM((64, D), jnp.float32),
    ],
)


def kernel(feature, edge_index, w):
    ridx = jax.random.randint(jax.random.key(1), (DROP,), 0, E)
    ridx2d = jnp.concatenate(
        [ridx, ridx[:RIDX_ROWS * 128 - DROP]]).reshape(RIDX_ROWS, 128)

    pad = jnp.arange(PADE, dtype=jnp.int32) % N
    srcp = jnp.concatenate([edge_index[0], pad]).reshape(R2D, 128)
    dstp = jnp.concatenate([edge_index[1], pad]).reshape(R2D, 128)
    w2d = w.reshape(W2DR, 128)

    pkK, wpK = _prep(srcp, dstp, w2d, ridx2d)
    p1 = _spmm(feature, pkK, wpK)
    h1 = _sum2(p1)
    p2 = _spmm(h1, pkK, wpK)
    return _final(feature, h1, p2)


# speedup vs baseline: 4.7331x; 1.0765x over previous
import jax
import jax.numpy as jnp
from jax import lax
from jax.experimental import pallas as pl
from jax.experimental.pallas import tpu as pltpu
from jax.experimental.pallas import tpu_sc as plsc

N = 10000
E = 320000
D = 128
DROP = int(0.7 * E)

NC, NS, L = 2, 16, 16
NW = NC * NS
RPW = 80
R2D = NW * RPW
E_PAD = R2D * 128
PADE = E_PAD - E
RPS = R2D // NS
NPT = 640
NACC = 10240
NROWS_T = NACC // NS
W2DR = E // 128
RIDX_ROWS = 1792
RIDX_PT = RIDX_ROWS // NS
CPW = 5248
KCH = CPW // 128
CBUF = CPW + L
FSUM = NW * 320

_SC_PARAMS = pltpu.CompilerParams(
    needs_layout_passes=False, use_tc_tiling_on_sc=False)
_MESH = plsc.VectorSubcoreMesh(core_axis_name="c", subcore_axis_name="s")


def _rsqrt_vec(x):
    y = 1.0 / x
    for _ in range(26):
        y = y * (1.5 - 0.5 * x * y * y)
    return y


def _prep_body(src_hbm, dst_hbm, w_hbm, ridx_hbm, pk_out, wp_out,
               src_v, dst_v, w_v, keep_v, ridx_v, pkc_v, wpc_v,
               onorm_v, inorm_v, tmp_v, ones_v, hsem,
               keep_s, hout_s, hin_s):
    c = lax.axis_index("c")
    s = lax.axis_index("s")
    wid = s * NC + c
    iota = lax.iota(jnp.int32, L)

    pltpu.sync_copy(src_hbm.at[pl.ds(s * RPS, RPS)], src_v)
    pltpu.sync_copy(dst_hbm.at[pl.ds(s * RPS, RPS)], dst_v)

    zv = jnp.zeros((L,), jnp.float32)
    ov = jnp.ones((L,), jnp.float32)
    for k in range(128 // L):
        ones_v[pl.ds(k * L, L)] = ov

    def _fill_zero(i, _):
        tmp_v[pl.ds(i * L, L)] = zv
        return 0
    lax.fori_loop(0, NPT // L, _fill_zero, 0)

    def _fill_one(i, _):
        keep_v[pl.ds(i * L, L)] = ov
        return 0
    lax.fori_loop(0, NACC // L, _fill_one, 0)

    pltpu.sync_copy(tmp_v, hout_s.at[pl.ds(s * NPT, NPT)])
    pltpu.sync_copy(tmp_v, hin_s.at[pl.ds(s * NPT, NPT)])
    kbase = s * (E_PAD // NS)
    pltpu.sync_copy(keep_v, keep_s.at[pl.ds(kbase, NACC)])
    pltpu.sync_copy(keep_v, keep_s.at[pl.ds(kbase + NACC, NACC)])
    plsc.subcore_barrier()

    pltpu.sync_copy(ridx_hbm.at[pl.ds(s * RIDX_PT, RIDX_PT)], ridx_v)

    def _row_drain():
        pltpu.make_async_copy(wp_out.at[pl.ds(0, 128)],
                              tmp_v.at[pl.ds(0, 128)], hsem).wait()

    def _drop_start(j):
        pltpu.async_copy(tmp_v.at[pl.ds(0, 128)],
                         keep_s.at[ridx_v.at[j]], hsem)

    for j0 in range(4):
        _drop_start(j0)

    def _drop(j, _):
        _row_drain()
        _drop_start(j)
        return 0
    lax.fori_loop(4, RIDX_PT, _drop, 0)
    for _ in range(4):
        _row_drain()

    def _hist_start(j):
        pltpu.async_copy(ones_v, hout_s.at[src_v.at[j]], hsem, add=True)
        pltpu.async_copy(ones_v, hin_s.at[dst_v.at[j]], hsem, add=True)

    for j0 in range(4):
        _hist_start(j0)

    def _hist(j, _):
        _row_drain()
        _row_drain()
        _hist_start(j)
        return 0
    lax.fori_loop(4, RPS, _hist, 0)
    for _ in range(8):
        _row_drain()
    plsc.subcore_barrier()

    for hist in (hout_s, hin_s):
        pltpu.sync_copy(hist.at[pl.ds(s * NPT, NPT)], tmp_v)

        def _norm(i, _):
            hv = tmp_v[pl.ds(i * L, L)]
            nvec = s * NPT + i * L + iota
            padc = jnp.where(nvec < PADE, 1.0, 0.0).astype(jnp.float32)
            deg = jnp.maximum(hv - padc, 1.0)
            tmp_v[pl.ds(i * L, L)] = _rsqrt_vec(deg)
            return 0
        lax.fori_loop(0, NPT // L, _norm, 0)
        pltpu.sync_copy(tmp_v, hist.at[pl.ds(s * NPT, NPT)])
    plsc.subcore_barrier()

    pltpu.sync_copy(hout_s, onorm_v)
    pltpu.sync_copy(hin_s, inorm_v)
    pltpu.sync_copy(keep_s.at[pl.ds(wid * (RPW * 128), RPW * 128)], keep_v)

    start = jnp.minimum(wid * RPW, W2DR - RPW)
    woff = wid * RPW - start
    pltpu.sync_copy(w_hbm.at[pl.ds(start, RPW)], w_v)

    def _prefill(i, _):
        v = (wid * CPW + i * L + iota) % N
        pkc_v[pl.ds(i * L, L)] = v | (v << 14)
        wpc_v[pl.ds(i * L, L)] = zv
        return 0
    lax.fori_loop(0, CBUF // L, _prefill, 0)

    loc = c * RPW

    def _wp(j, off):
        jr = jnp.minimum(woff + j, RPW - 1)
        for k in range(128 // L):
            sl = pl.ds(k * L, L)
            sidx = src_v[loc + j, sl]
            didx = dst_v[loc + j, sl]
            on = plsc.load_gather(onorm_v, [sidx])
            inr = plsc.load_gather(inorm_v, [didx])
            kv = keep_v[pl.ds(j * 128 + k * L, L)]
            wv = w_v[jr, sl]
            eid = (wid * RPW + j) * 128 + k * L + iota
            m = (kv != 0.0) & (eid < E)
            wp = wv * kv * on * inr
            pk = sidx | (didx << 14)
            plsc.store_compressed(pkc_v.at[pl.ds(off, L)], pk, mask=m)
            plsc.store_compressed(wpc_v.at[pl.ds(off, L)], wp, mask=m)
            off = off + plsc.all_reduce_population_count(m)[0]
        return off
    lax.fori_loop(0, RPW, _wp, jnp.int32(0))

    pltpu.sync_copy(pkc_v.at[pl.ds(0, CPW)], pk_out.at[pl.ds(wid * CPW, CPW)])
    pltpu.sync_copy(wpc_v.at[pl.ds(0, CPW)], wp_out.at[pl.ds(wid * CPW, CPW)])


_prep = pl.kernel(
    _prep_body,
    out_type=(jax.ShapeDtypeStruct((NW * CPW,), jnp.int32),
              jax.ShapeDtypeStruct((NW * CPW,), jnp.float32)),
    compiler_params=_SC_PARAMS,
    mesh=_MESH,
    scratch_types=[
        pltpu.VMEM((RPS, 128), jnp.int32),
        pltpu.VMEM((RPS, 128), jnp.int32),
        pltpu.VMEM((RPW, 128), jnp.float32),
        pltpu.VMEM((NACC,), jnp.float32),
        pltpu.VMEM((RIDX_PT, 128), jnp.int32),
        pltpu.VMEM((CBUF,), jnp.int32),
        pltpu.VMEM((CBUF,), jnp.float32),
        pltpu.VMEM((NACC,), jnp.float32),
        pltpu.VMEM((NACC,), jnp.float32),
        pltpu.VMEM((NPT,), jnp.float32),
        pltpu.VMEM((128,), jnp.float32),
        pltpu.SemaphoreType.DMA,
        pltpu.VMEM_SHARED((E_PAD,), jnp.float32),
        pltpu.VMEM_SHARED((NACC,), jnp.float32),
        pltpu.VMEM_SHARED((NACC,), jnp.float32),
    ],
)


def _spmm_body(tbl, pk_hbm, wp_hbm, p_hbm,
               pk_v, wp_v, sidx_v, didx_v, rb0, rb1, gsem, ssem, acc_s):
    c = lax.axis_index("c")
    s = lax.axis_index("s")
    wid = s * NC + c
    bufs = (rb0, rb1)

    pltpu.sync_copy(pk_hbm.at[pl.ds(wid * CPW, CPW)], pk_v)
    pltpu.sync_copy(wp_hbm.at[pl.ds(wid * CPW, CPW)], wp_v)

    zv = jnp.zeros((L,), jnp.float32)

    def _zrow(i, _):
        for k in range(D // L):
            rb0[i, pl.ds(k * L, L)] = zv
        return 0
    lax.fori_loop(0, 128, _zrow, 0)
    for q in range(NROWS_T // 128):
        pltpu.sync_copy(rb0, acc_s.at[pl.ds(s * NROWS_T + q * 128, 128)])
    plsc.subcore_barrier()

    def _gather_start(j, b):
        for k in range(128 // L):
            sl = pl.ds(k * L, L)
            pkv = pk_v[pl.ds(j * 128 + k * L, L)]
            sidx_v[b, sl] = pkv & 0x3FFF
            didx_v[b, sl] = pkv >> 14
        pltpu.async_copy(tbl.at[sidx_v.at[b]], bufs[b], gsem)

    def _gather_wait(b):
        pltpu.make_async_copy(tbl.at[pl.ds(0, 128)], bufs[b], gsem).wait()

    def _scatter_start(b):
        pltpu.async_copy(bufs[b], acc_s.at[didx_v.at[b]], ssem, add=True)

    def _scatter_drain():
        pltpu.make_async_copy(tbl.at[pl.ds(0, 128)], rb0, ssem).wait()

    def _scale(j, b):
        buf = bufs[b]

        def _grp(g, _):
            wv = wp_v[pl.ds(j * 128 + g * L, L)]
            for rr in range(L):
                wsc = wv[rr]
                for k in range(D // L):
                    sl = pl.ds(k * L, L)
                    buf[g * L + rr, sl] = buf[g * L + rr, sl] * wsc
            return 0
        lax.fori_loop(0, 128 // L, _grp, 0)

    def _sub(j, b, drain, gnext):
        _gather_wait(b)
        _scale(j, b)
        _scatter_start(b)
        if drain:
            _scatter_drain()
        if gnext:
            _gather_start(j + 1, 1 - b)

    _gather_start(0, 0)
    _sub(0, 0, False, True)

    def _outer(jo, _):
        j = 1 + 2 * jo
        _sub(j, 1, True, True)
        _sub(j + 1, 0, True, True)
        return 0
    lax.fori_loop(0, (KCH - 3) // 2, _outer, 0)

    _sub(KCH - 2, 1, True, True)
    _sub(KCH - 1, 0, True, False)
    _scatter_drain()
    plsc.subcore_barrier()

    pltpu.sync_copy(acc_s.at[pl.ds(s * NROWS_T, NROWS_T)],
                    p_hbm.at[c, pl.ds(s * NROWS_T, NROWS_T)])


_spmm = pl.kernel(
    _spmm_body,
    out_type=jax.ShapeDtypeStruct((NC, NACC, D), jnp.float32),
    compiler_params=_SC_PARAMS,
    mesh=_MESH,
    scratch_types=[
        pltpu.VMEM((CPW,), jnp.int32),
        pltpu.VMEM((CPW,), jnp.float32),
        pltpu.VMEM((2, 128), jnp.int32),
        pltpu.VMEM((2, 128), jnp.int32),
        pltpu.VMEM((128, D), jnp.float32),
        pltpu.VMEM((128, D), jnp.float32),
        pltpu.SemaphoreType.DMA,
        pltpu.SemaphoreType.DMA,
        pltpu.VMEM_SHARED((NACC, D), jnp.float32),
    ],
)


def _sum2_body(p_hbm, h_hbm, a_v, b_v):
    c = lax.axis_index("c")
    s = lax.axis_index("s")
    wid = s * NC + c
    base = wid * (NACC // NW)

    def _chunk(q, _):
        r = base + q * 64
        pltpu.sync_copy(p_hbm.at[0, pl.ds(r, 64)], a_v)
        pltpu.sync_copy(p_hbm.at[1, pl.ds(r, 64)], b_v)

        def _row(i, _):
            for k in range(D // L):
                sl = pl.ds(k * L, L)
                a_v[i, sl] = a_v[i, sl] + b_v[i, sl]
            return 0
        lax.fori_loop(0, 64, _row, 0)
        pltpu.sync_copy(a_v, h_hbm.at[pl.ds(r, 64)])
        return 0
    lax.fori_loop(0, (NACC // NW) // 64, _chunk, 0)


_sum2 = pl.kernel(
    _sum2_body,
    out_type=jax.ShapeDtypeStruct((NACC, D), jnp.float32),
    compiler_params=_SC_PARAMS,
    mesh=_MESH,
    scratch_types=[
        pltpu.VMEM((64, D), jnp.float32),
        pltpu.VMEM((64, D), jnp.float32),
    ],
)


def _final_body(f_hbm, h1_hbm, p2_hbm, o_hbm, a_v, b_v):
    c = lax.axis_index("c")
    s = lax.axis_index("s")
    wid = s * NC + c
    base = jnp.minimum(wid * 320, N - 320)

    def _chunk(q, _):
        r = base + q * 64
        pltpu.sync_copy(h1_hbm.at[pl.ds(r, 64)], a_v)
        pltpu.sync_copy(p2_hbm.at[0, pl.ds(r, 64)], b_v)

        def _row1(i, _):
            for k in range(D // L):
                sl = pl.ds(k * L, L)
                a_v[i, sl] = a_v[i, sl] + b_v[i, sl]
            return 0
        lax.fori_loop(0, 64, _row1, 0)
        pltpu.sync_copy(p2_hbm.at[1, pl.ds(r, 64)], b_v)

        def _row2(i, _):
            for k in range(D // L):
                sl = pl.ds(k * L, L)
                a_v[i, sl] = a_v[i, sl] + b_v[i, sl]
            return 0
        lax.fori_loop(0, 64, _row2, 0)
        pltpu.sync_copy(f_hbm.at[pl.ds(r, 64)], b_v)

        def _row3(i, _):
            for k in range(D // L):
                sl = pl.ds(k * L, L)
                a_v[i, sl] = (a_v[i, sl] + b_v[i, sl]) * (1.0 / 3.0)
            return 0
        lax.fori_loop(0, 64, _row3, 0)
        pltpu.sync_copy(a_v, o_hbm.at[pl.ds(r, 64)])
        return 0
    lax.fori_loop(0, 5, _chunk, 0)


_final = pl.kernel(
    _final_body,
    out_type=jax.ShapeDtypeStruct((N, D), jnp.float32),
    compiler_params=_SC_PARAMS,
    mesh=_MESH,
    scratch_types=[
        pltpu.VMEM((64, D), jnp.float32),
        pltpu.VMEM((64, D), jnp.float32),
    ],
)


def kernel(feature, edge_index, w):
    ridx = jax.random.randint(jax.random.key(1), (DROP,), 0, E)
    ridx2d = jnp.concatenate(
        [ridx, ridx[:RIDX_ROWS * 128 - DROP]]).reshape(RIDX_ROWS, 128)

    pad = jnp.arange(PADE, dtype=jnp.int32) % N
    srcp = jnp.concatenate([edge_index[0], pad]).reshape(R2D, 128)
    dstp = jnp.concatenate([edge_index[1], pad]).reshape(R2D, 128)
    w2d = w.reshape(W2DR, 128)

    pkK, wpK = _prep(srcp, dstp, w2d, ridx2d)
    p1 = _spmm(feature, pkK, wpK)
    h1 = _sum2(p1)
    p2 = _spmm(h1, pkK, wpK)
    return _final(feature, h1, p2)


# speedup vs baseline: 4.9510x; 1.0460x over previous
import jax
import jax.numpy as jnp
from jax import lax
from jax.experimental import pallas as pl
from jax.experimental.pallas import tpu as pltpu
from jax.experimental.pallas import tpu_sc as plsc

N = 10000
E = 320000
D = 128
DROP = int(0.7 * E)

NC, NS, L = 2, 16, 16
NW = NC * NS
RPW = 80
R2D = NW * RPW
E_PAD = R2D * 128
PADE = E_PAD - E
RPS = R2D // NS
NPT = 640
NACC = 10240
NROWS_T = NACC // NS
W2DR = E // 128
RIDX_ROWS = 1792
RIDX_PT = RIDX_ROWS // NS
CPW = 5248
KCH = CPW // 128
CBUF = CPW + L
FSUM = NW * 320

_SC_PARAMS = pltpu.CompilerParams(
    needs_layout_passes=False, use_tc_tiling_on_sc=False)
_MESH = plsc.VectorSubcoreMesh(core_axis_name="c", subcore_axis_name="s")


def _rsqrt_vec(x):
    y = 1.0 / x
    for _ in range(26):
        y = y * (1.5 - 0.5 * x * y * y)
    return y


def _prep_body(src_hbm, dst_hbm, w_hbm, ridx_hbm, pk_out, wp_out,
               src_v, dst_v, w_v, keep_v, ridx_v, pkc_v, wpc_v,
               onorm_v, inorm_v, tmp_v, ones_v, hsem,
               keep_s, hout_s, hin_s):
    c = lax.axis_index("c")
    s = lax.axis_index("s")
    wid = s * NC + c
    iota = lax.iota(jnp.int32, L)

    pltpu.sync_copy(src_hbm.at[pl.ds(s * RPS, RPS)], src_v)
    pltpu.sync_copy(dst_hbm.at[pl.ds(s * RPS, RPS)], dst_v)

    zv = jnp.zeros((L,), jnp.float32)
    ov = jnp.ones((L,), jnp.float32)
    for k in range(128 // L):
        ones_v[pl.ds(k * L, L)] = ov

    def _fill_zero(i, _):
        tmp_v[pl.ds(i * L, L)] = zv
        return 0
    lax.fori_loop(0, NPT // L, _fill_zero, 0)

    def _fill_one(i, _):
        keep_v[pl.ds(i * L, L)] = ov
        return 0
    lax.fori_loop(0, NACC // L, _fill_one, 0)

    pltpu.sync_copy(tmp_v, hout_s.at[pl.ds(s * NPT, NPT)])
    pltpu.sync_copy(tmp_v, hin_s.at[pl.ds(s * NPT, NPT)])
    kbase = s * (E_PAD // NS)
    pltpu.sync_copy(keep_v, keep_s.at[pl.ds(kbase, NACC)])
    pltpu.sync_copy(keep_v, keep_s.at[pl.ds(kbase + NACC, NACC)])
    plsc.subcore_barrier()

    pltpu.sync_copy(ridx_hbm.at[pl.ds(s * RIDX_PT, RIDX_PT)], ridx_v)

    def _row_drain():
        pltpu.make_async_copy(wp_out.at[pl.ds(0, 128)],
                              tmp_v.at[pl.ds(0, 128)], hsem).wait()

    def _drop_start(j):
        pltpu.async_copy(tmp_v.at[pl.ds(0, 128)],
                         keep_s.at[ridx_v.at[j]], hsem)

    for j0 in range(4):
        _drop_start(j0)

    def _drop(j, _):
        _row_drain()
        _drop_start(j)
        return 0
    lax.fori_loop(4, RIDX_PT, _drop, 0)
    for _ in range(4):
        _row_drain()

    def _hist_start(j):
        pltpu.async_copy(ones_v, hout_s.at[src_v.at[j]], hsem, add=True)
        pltpu.async_copy(ones_v, hin_s.at[dst_v.at[j]], hsem, add=True)

    for j0 in range(4):
        _hist_start(j0)

    def _hist(j, _):
        _row_drain()
        _row_drain()
        _hist_start(j)
        return 0
    lax.fori_loop(4, RPS, _hist, 0)
    for _ in range(8):
        _row_drain()
    plsc.subcore_barrier()

    for hist in (hout_s, hin_s):
        pltpu.sync_copy(hist.at[pl.ds(s * NPT, NPT)], tmp_v)

        def _norm(i, _):
            hv = tmp_v[pl.ds(i * L, L)]
            nvec = s * NPT + i * L + iota
            padc = jnp.where(nvec < PADE, 1.0, 0.0).astype(jnp.float32)
            deg = jnp.maximum(hv - padc, 1.0)
            tmp_v[pl.ds(i * L, L)] = _rsqrt_vec(deg)
            return 0
        lax.fori_loop(0, NPT // L, _norm, 0)
        pltpu.sync_copy(tmp_v, hist.at[pl.ds(s * NPT, NPT)])
    plsc.subcore_barrier()

    pltpu.sync_copy(hout_s, onorm_v)
    pltpu.sync_copy(hin_s, inorm_v)
    pltpu.sync_copy(keep_s.at[pl.ds(wid * (RPW * 128), RPW * 128)], keep_v)

    start = jnp.minimum(wid * RPW, W2DR - RPW)
    woff = wid * RPW - start
    pltpu.sync_copy(w_hbm.at[pl.ds(start, RPW)], w_v)

    def _prefill(i, _):
        v = (wid * CPW + i * L + iota) % N
        pkc_v[pl.ds(i * L, L)] = v | (v << 14)
        wpc_v[pl.ds(i * L, L)] = zv
        return 0
    lax.fori_loop(0, CBUF // L, _prefill, 0)

    loc = c * RPW

    def _wp(j, off):
        jr = jnp.minimum(woff + j, RPW - 1)
        for k in range(128 // L):
            sl = pl.ds(k * L, L)
            sidx = src_v[loc + j, sl]
            didx = dst_v[loc + j, sl]
            on = plsc.load_gather(onorm_v, [sidx])
            inr = plsc.load_gather(inorm_v, [didx])
            kv = keep_v[pl.ds(j * 128 + k * L, L)]
            wv = w_v[jr, sl]
            eid = (wid * RPW + j) * 128 + k * L + iota
            m = (kv != 0.0) & (eid < E)
            wp = wv * kv * on * inr
            pk = sidx | (didx << 14)
            plsc.store_compressed(pkc_v.at[pl.ds(off, L)], pk, mask=m)
            plsc.store_compressed(wpc_v.at[pl.ds(off, L)], wp, mask=m)
            off = off + plsc.all_reduce_population_count(m)[0]
        return off
    lax.fori_loop(0, RPW, _wp, jnp.int32(0))

    pltpu.sync_copy(pkc_v.at[pl.ds(0, CPW)], pk_out.at[pl.ds(wid * CPW, CPW)])
    pltpu.sync_copy(wpc_v.at[pl.ds(0, CPW)], wp_out.at[pl.ds(wid * CPW, CPW)])


_prep = pl.kernel(
    _prep_body,
    out_type=(jax.ShapeDtypeStruct((NW * CPW,), jnp.int32),
              jax.ShapeDtypeStruct((NW * CPW,), jnp.float32)),
    compiler_params=_SC_PARAMS,
    mesh=_MESH,
    scratch_types=[
        pltpu.VMEM((RPS, 128), jnp.int32),
        pltpu.VMEM((RPS, 128), jnp.int32),
        pltpu.VMEM((RPW, 128), jnp.float32),
        pltpu.VMEM((NACC,), jnp.float32),
        pltpu.VMEM((RIDX_PT, 128), jnp.int32),
        pltpu.VMEM((CBUF,), jnp.int32),
        pltpu.VMEM((CBUF,), jnp.float32),
        pltpu.VMEM((NACC,), jnp.float32),
        pltpu.VMEM((NACC,), jnp.float32),
        pltpu.VMEM((NPT,), jnp.float32),
        pltpu.VMEM((128,), jnp.float32),
        pltpu.SemaphoreType.DMA,
        pltpu.VMEM_SHARED((E_PAD,), jnp.float32),
        pltpu.VMEM_SHARED((NACC,), jnp.float32),
        pltpu.VMEM_SHARED((NACC,), jnp.float32),
    ],
)


def _spmm_body(tbl, pk_hbm, wp_hbm, p_hbm,
               pk_v, wp_v, sidx_v, didx_v, rb0, rb1, gsem, ssem, acc_s):
    c = lax.axis_index("c")
    s = lax.axis_index("s")
    wid = s * NC + c
    bufs = (rb0, rb1)

    pltpu.sync_copy(pk_hbm.at[pl.ds(wid * CPW, CPW)], pk_v)
    pltpu.sync_copy(wp_hbm.at[pl.ds(wid * CPW, CPW)], wp_v)

    zv = jnp.zeros((L,), jnp.float32)

    def _zrow(i, _):
        for k in range(D // L):
            rb0[i, pl.ds(k * L, L)] = zv
        return 0
    lax.fori_loop(0, 128, _zrow, 0)
    for q in range(NROWS_T // 128):
        pltpu.sync_copy(rb0, acc_s.at[pl.ds(s * NROWS_T + q * 128, 128)])
    plsc.subcore_barrier()

    def _gather_start(j, b):
        for k in range(128 // L):
            sl = pl.ds(k * L, L)
            pkv = pk_v[pl.ds(j * 128 + k * L, L)]
            sidx_v[b, sl] = pkv & 0x3FFF
            didx_v[b, sl] = pkv >> 14
        pltpu.async_copy(tbl.at[sidx_v.at[b]], bufs[b], gsem)

    def _gather_wait(b):
        pltpu.make_async_copy(tbl.at[pl.ds(0, 128)], bufs[b], gsem).wait()

    def _scatter_start(b):
        pltpu.async_copy(bufs[b], acc_s.at[didx_v.at[b]], ssem, add=True)

    def _scatter_drain():
        pltpu.make_async_copy(tbl.at[pl.ds(0, 128)], rb0, ssem).wait()

    def _scale(j, b):
        buf = bufs[b]

        def _grp(g, _):
            wv = wp_v[pl.ds(j * 128 + g * L, L)]
            for rr in range(L):
                wsc = wv[rr]
                for k in range(D // L):
                    sl = pl.ds(k * L, L)
                    buf[g * L + rr, sl] = buf[g * L + rr, sl] * wsc
            return 0
        lax.fori_loop(0, 128 // L, _grp, 0)

    def _sub(j, b, drain, gnext):
        _gather_wait(b)
        _scale(j, b)
        _scatter_start(b)
        if drain:
            _scatter_drain()
        if gnext:
            _gather_start(j + 1, 1 - b)

    _gather_start(0, 0)
    _sub(0, 0, False, True)

    def _outer(jo, _):
        j = 1 + 2 * jo
        _sub(j, 1, True, True)
        _sub(j + 1, 0, True, True)
        return 0
    lax.fori_loop(0, (KCH - 3) // 2, _outer, 0)

    _sub(KCH - 2, 1, True, True)
    _sub(KCH - 1, 0, True, False)
    _scatter_drain()
    plsc.subcore_barrier()

    pltpu.sync_copy(acc_s.at[pl.ds(s * NROWS_T, NROWS_T)],
                    p_hbm.at[c, pl.ds(s * NROWS_T, NROWS_T)])


_spmm = pl.kernel(
    _spmm_body,
    out_type=jax.ShapeDtypeStruct((NC, NACC, D), jnp.float32),
    compiler_params=_SC_PARAMS,
    mesh=_MESH,
    scratch_types=[
        pltpu.VMEM((CPW,), jnp.int32),
        pltpu.VMEM((CPW,), jnp.float32),
        pltpu.VMEM((2, 128), jnp.int32),
        pltpu.VMEM((2, 128), jnp.int32),
        pltpu.VMEM((128, D), jnp.float32),
        pltpu.VMEM((128, D), jnp.float32),
        pltpu.SemaphoreType.DMA,
        pltpu.SemaphoreType.DMA,
        pltpu.VMEM_SHARED((NACC, D), jnp.float32),
    ],
)


def _sum2_body(p_hbm, h_hbm, a_v, b_v, lsem):
    c = lax.axis_index("c")
    s = lax.axis_index("s")
    wid = s * NC + c
    base = wid * (NACC // NW)

    def _chunk(q, _):
        r = base + q * 64
        pltpu.async_copy(p_hbm.at[0, pl.ds(r, 64)], a_v, lsem)
        pltpu.async_copy(p_hbm.at[1, pl.ds(r, 64)], b_v, lsem)
        pltpu.make_async_copy(p_hbm.at[0, pl.ds(0, 64)], a_v, lsem).wait()
        pltpu.make_async_copy(p_hbm.at[0, pl.ds(0, 64)], b_v, lsem).wait()

        def _row(i, _):
            for k in range(D // L):
                sl = pl.ds(k * L, L)
                a_v[i, sl] = a_v[i, sl] + b_v[i, sl]
            return 0
        lax.fori_loop(0, 64, _row, 0)
        pltpu.sync_copy(a_v, h_hbm.at[pl.ds(r, 64)])
        return 0
    lax.fori_loop(0, (NACC // NW) // 64, _chunk, 0)


_sum2 = pl.kernel(
    _sum2_body,
    out_type=jax.ShapeDtypeStruct((NACC, D), jnp.float32),
    compiler_params=_SC_PARAMS,
    mesh=_MESH,
    scratch_types=[
        pltpu.VMEM((64, D), jnp.float32),
        pltpu.VMEM((64, D), jnp.float32),
        pltpu.SemaphoreType.DMA,
    ],
)


def _final_body(f_hbm, h1_hbm, p2_hbm, o_hbm, a_v, b_v, c_v, d_v, lsem):
    c = lax.axis_index("c")
    s = lax.axis_index("s")
    wid = s * NC + c
    base = jnp.minimum(wid * 320, N - 320)

    def _chunk(q, _):
        r = base + q * 64
        pltpu.async_copy(h1_hbm.at[pl.ds(r, 64)], a_v, lsem)
        pltpu.async_copy(p2_hbm.at[0, pl.ds(r, 64)], b_v, lsem)
        pltpu.async_copy(p2_hbm.at[1, pl.ds(r, 64)], c_v, lsem)
        pltpu.async_copy(f_hbm.at[pl.ds(r, 64)], d_v, lsem)
        for _ in range(4):
            pltpu.make_async_copy(
                h1_hbm.at[pl.ds(0, 64)], a_v, lsem).wait()

        def _row(i, _):
            for k in range(D // L):
                sl = pl.ds(k * L, L)
                a_v[i, sl] = (a_v[i, sl] + b_v[i, sl]
                              + c_v[i, sl] + d_v[i, sl]) * (1.0 / 3.0)
            return 0
        lax.fori_loop(0, 64, _row, 0)
        pltpu.sync_copy(a_v, o_hbm.at[pl.ds(r, 64)])
        return 0
    lax.fori_loop(0, 5, _chunk, 0)


_final = pl.kernel(
    _final_body,
    out_type=jax.ShapeDtypeStruct((N, D), jnp.float32),
    compiler_params=_SC_PARAMS,
    mesh=_MESH,
    scratch_types=[
        pltpu.VMEM((64, D), jnp.float32),
        pltpu.VMEM((64, D), jnp.float32),
        pltpu.VMEM((64, D), jnp.float32),
        pltpu.VMEM((64, D), jnp.float32),
        pltpu.SemaphoreType.DMA,
    ],
)


def kernel(feature, edge_index, w):
    ridx = jax.random.randint(jax.random.key(1), (DROP,), 0, E)
    ridx2d = jnp.concatenate(
        [ridx, ridx[:RIDX_ROWS * 128 - DROP]]).reshape(RIDX_ROWS, 128)

    pad = jnp.arange(PADE, dtype=jnp.int32) % N
    srcp = jnp.concatenate([edge_index[0], pad]).reshape(R2D, 128)
    dstp = jnp.concatenate([edge_index[1], pad]).reshape(R2D, 128)
    w2d = w.reshape(W2DR, 128)

    pkK, wpK = _prep(srcp, dstp, w2d, ridx2d)
    p1 = _spmm(feature, pkK, wpK)
    h1 = _sum2(p1)
    p2 = _spmm(h1, pkK, wpK)
    return _final(feature, h1, p2)


# speedup vs baseline: 5.6599x; 1.1432x over previous
import jax
import jax.numpy as jnp
from jax import lax
from jax.experimental import pallas as pl
from jax.experimental.pallas import tpu as pltpu
from jax.experimental.pallas import tpu_sc as plsc

N = 10000
E = 320000
D = 128
DROP = int(0.7 * E)

NC, NS, L = 2, 16, 16
NW = NC * NS
RPW = 80
R2D = NW * RPW
E_PAD = R2D * 128
PADE = E_PAD - E
RPS = R2D // NS
NPT = 640
NACC = 10240
NROWS_T = NACC // NS
W2DR = E // 128
RIDX_ROWS = 1792
RIDX_PT = RIDX_ROWS // NS
CPW = 5248
KCH = CPW // 128
CBUF = CPW + L
FSUM = NW * 320

_SC_PARAMS = pltpu.CompilerParams(
    needs_layout_passes=False, use_tc_tiling_on_sc=False)
_MESH = plsc.VectorSubcoreMesh(core_axis_name="c", subcore_axis_name="s")


def _rsqrt_vec(x):
    y = 1.0 / x
    for _ in range(26):
        y = y * (1.5 - 0.5 * x * y * y)
    return y


def _prep_body(src_hbm, dst_hbm, w_hbm, ridx_hbm, pk_out, wp_out,
               src_v, dst_v, w_v, keep_v, ridx_v, pkc_v, wpc_v,
               onorm_v, inorm_v, tmp_v, ones_v, hsem,
               keep_s, hout_s, hin_s):
    c = lax.axis_index("c")
    s = lax.axis_index("s")
    wid = s * NC + c
    iota = lax.iota(jnp.int32, L)

    pltpu.sync_copy(src_hbm.at[pl.ds(s * RPS, RPS)], src_v)
    pltpu.sync_copy(dst_hbm.at[pl.ds(s * RPS, RPS)], dst_v)

    zv = jnp.zeros((L,), jnp.float32)
    ov = jnp.ones((L,), jnp.float32)
    for k in range(128 // L):
        ones_v[pl.ds(k * L, L)] = ov

    def _fill_zero(i, _):
        tmp_v[pl.ds(i * L, L)] = zv
        return 0
    lax.fori_loop(0, NPT // L, _fill_zero, 0)

    def _fill_one(i, _):
        keep_v[pl.ds(i * L, L)] = ov
        return 0
    lax.fori_loop(0, NACC // L, _fill_one, 0)

    pltpu.sync_copy(tmp_v, hout_s.at[pl.ds(s * NPT, NPT)])
    pltpu.sync_copy(tmp_v, hin_s.at[pl.ds(s * NPT, NPT)])
    kbase = s * (E_PAD // NS)
    pltpu.sync_copy(keep_v, keep_s.at[pl.ds(kbase, NACC)])
    pltpu.sync_copy(keep_v, keep_s.at[pl.ds(kbase + NACC, NACC)])
    plsc.subcore_barrier()

    pltpu.sync_copy(ridx_hbm.at[pl.ds(s * RIDX_PT, RIDX_PT)], ridx_v)

    def _row_drain():
        pltpu.make_async_copy(wp_out.at[pl.ds(0, 128)],
                              tmp_v.at[pl.ds(0, 128)], hsem).wait()

    def _drop_start(j):
        pltpu.async_copy(tmp_v.at[pl.ds(0, 128)],
                         keep_s.at[ridx_v.at[j]], hsem)

    for j0 in range(4):
        _drop_start(j0)

    def _drop(j, _):
        _row_drain()
        _drop_start(j)
        return 0
    lax.fori_loop(4, RIDX_PT, _drop, 0)
    for _ in range(4):
        _row_drain()

    def _hist_start(j):
        pltpu.async_copy(ones_v, hout_s.at[src_v.at[j]], hsem, add=True)
        pltpu.async_copy(ones_v, hin_s.at[dst_v.at[j]], hsem, add=True)

    for j0 in range(4):
        _hist_start(j0)

    def _hist(j, _):
        _row_drain()
        _row_drain()
        _hist_start(j)
        return 0
    lax.fori_loop(4, RPS, _hist, 0)
    for _ in range(8):
        _row_drain()
    plsc.subcore_barrier()

    for hist in (hout_s, hin_s):
        pltpu.sync_copy(hist.at[pl.ds(s * NPT, NPT)], tmp_v)

        def _norm(i, _):
            hv = tmp_v[pl.ds(i * L, L)]
            nvec = s * NPT + i * L + iota
            padc = jnp.where(nvec < PADE, 1.0, 0.0).astype(jnp.float32)
            deg = jnp.maximum(hv - padc, 1.0)
            tmp_v[pl.ds(i * L, L)] = _rsqrt_vec(deg)
            return 0
        lax.fori_loop(0, NPT // L, _norm, 0)
        pltpu.sync_copy(tmp_v, hist.at[pl.ds(s * NPT, NPT)])
    plsc.subcore_barrier()

    pltpu.sync_copy(hout_s, onorm_v)
    pltpu.sync_copy(hin_s, inorm_v)
    pltpu.sync_copy(keep_s.at[pl.ds(wid * (RPW * 128), RPW * 128)], keep_v)

    start = jnp.minimum(wid * RPW, W2DR - RPW)
    woff = wid * RPW - start
    pltpu.sync_copy(w_hbm.at[pl.ds(start, RPW)], w_v)

    def _prefill(i, _):
        v = (wid * CPW + i * L + iota) % N
        pkc_v[pl.ds(i * L, L)] = v | (v << 14)
        wpc_v[pl.ds(i * L, L)] = zv
        return 0
    lax.fori_loop(0, CBUF // L, _prefill, 0)

    loc = c * RPW

    def _wp(j, off):
        jr = jnp.minimum(woff + j, RPW - 1)
        for k in range(128 // L):
            sl = pl.ds(k * L, L)
            sidx = src_v[loc + j, sl]
            didx = dst_v[loc + j, sl]
            on = plsc.load_gather(onorm_v, [sidx])
            inr = plsc.load_gather(inorm_v, [didx])
            kv = keep_v[pl.ds(j * 128 + k * L, L)]
            wv = w_v[jr, sl]
            eid = (wid * RPW + j) * 128 + k * L + iota
            m = (kv != 0.0) & (eid < E)
            wp = wv * kv * on * inr
            pk = sidx | (didx << 14)
            plsc.store_compressed(pkc_v.at[pl.ds(off, L)], pk, mask=m)
            plsc.store_compressed(wpc_v.at[pl.ds(off, L)], wp, mask=m)
            off = off + plsc.all_reduce_population_count(m)[0]
        return off
    lax.fori_loop(0, RPW, _wp, jnp.int32(0))

    pltpu.sync_copy(pkc_v.at[pl.ds(0, CPW)], pk_out.at[pl.ds(wid * CPW, CPW)])
    pltpu.sync_copy(wpc_v.at[pl.ds(0, CPW)], wp_out.at[pl.ds(wid * CPW, CPW)])


_prep = pl.kernel(
    _prep_body,
    out_type=(jax.ShapeDtypeStruct((NW * CPW,), jnp.int32),
              jax.ShapeDtypeStruct((NW * CPW,), jnp.float32)),
    compiler_params=_SC_PARAMS,
    mesh=_MESH,
    scratch_types=[
        pltpu.VMEM((RPS, 128), jnp.int32),
        pltpu.VMEM((RPS, 128), jnp.int32),
        pltpu.VMEM((RPW, 128), jnp.float32),
        pltpu.VMEM((NACC,), jnp.float32),
        pltpu.VMEM((RIDX_PT, 128), jnp.int32),
        pltpu.VMEM((CBUF,), jnp.int32),
        pltpu.VMEM((CBUF,), jnp.float32),
        pltpu.VMEM((NACC,), jnp.float32),
        pltpu.VMEM((NACC,), jnp.float32),
        pltpu.VMEM((NPT,), jnp.float32),
        pltpu.VMEM((128,), jnp.float32),
        pltpu.SemaphoreType.DMA,
        pltpu.VMEM_SHARED((E_PAD,), jnp.float32),
        pltpu.VMEM_SHARED((NACC,), jnp.float32),
        pltpu.VMEM_SHARED((NACC,), jnp.float32),
    ],
)


def _spmm_body(tbl, pk_hbm, wp_hbm, p_hbm,
               pk_v, wp_v, sidx_v, didx_v, rb0, rb1, gsem, ssem, acc_s):
    c = lax.axis_index("c")
    s = lax.axis_index("s")
    wid = s * NC + c
    bufs = (rb0, rb1)

    pltpu.sync_copy(pk_hbm.at[pl.ds(wid * CPW, CPW)], pk_v)
    pltpu.sync_copy(wp_hbm.at[pl.ds(wid * CPW, CPW)], wp_v)

    zv = jnp.zeros((L,), jnp.float32)

    def _zrow(i, _):
        for k in range(D // L):
            rb0[i, pl.ds(k * L, L)] = zv
        return 0
    lax.fori_loop(0, 128, _zrow, 0)
    for q in range(NROWS_T // 128):
        pltpu.sync_copy(rb0, acc_s.at[pl.ds(s * NROWS_T + q * 128, 128)])
    plsc.subcore_barrier()

    def _gather_start(j, b):
        for k in range(128 // L):
            sl = pl.ds(k * L, L)
            pkv = pk_v[pl.ds(j * 128 + k * L, L)]
            sidx_v[b, sl] = pkv & 0x3FFF
            didx_v[b, sl] = pkv >> 14
        pltpu.async_copy(tbl.at[sidx_v.at[b]], bufs[b], gsem)

    def _gather_wait(b):
        pltpu.make_async_copy(tbl.at[pl.ds(0, 128)], bufs[b], gsem).wait()

    def _scatter_start(b):
        pltpu.async_copy(bufs[b], acc_s.at[didx_v.at[b]], ssem, add=True)

    def _scatter_drain():
        pltpu.make_async_copy(tbl.at[pl.ds(0, 128)], rb0, ssem).wait()

    def _scale(j, b):
        buf = bufs[b]

        def _grp(g, _):
            wv = wp_v[pl.ds(j * 128 + g * L, L)]
            for rr in range(L):
                wsc = wv[rr]
                for k in range(D // L):
                    sl = pl.ds(k * L, L)
                    buf[g * L + rr, sl] = buf[g * L + rr, sl] * wsc
            return 0
        lax.fori_loop(0, 128 // L, _grp, 0)

    def _sub(j, b, drain, gnext):
        _gather_wait(b)
        if drain:
            _scatter_drain()
        if gnext:
            _gather_start(j + 1, 1 - b)
        _scale(j, b)
        _scatter_start(b)

    _gather_start(0, 0)
    _sub(0, 0, False, True)

    def _outer(jo, _):
        j = 1 + 2 * jo
        _sub(j, 1, True, True)
        _sub(j + 1, 0, True, True)
        return 0
    lax.fori_loop(0, (KCH - 3) // 2, _outer, 0)

    _sub(KCH - 2, 1, True, True)
    _sub(KCH - 1, 0, True, False)
    _scatter_drain()
    plsc.subcore_barrier()

    pltpu.sync_copy(acc_s.at[pl.ds(s * NROWS_T, NROWS_T)],
                    p_hbm.at[c, pl.ds(s * NROWS_T, NROWS_T)])


_spmm = pl.kernel(
    _spmm_body,
    out_type=jax.ShapeDtypeStruct((NC, NACC, D), jnp.float32),
    compiler_params=_SC_PARAMS,
    mesh=_MESH,
    scratch_types=[
        pltpu.VMEM((CPW,), jnp.int32),
        pltpu.VMEM((CPW,), jnp.float32),
        pltpu.VMEM((2, 128), jnp.int32),
        pltpu.VMEM((2, 128), jnp.int32),
        pltpu.VMEM((128, D), jnp.float32),
        pltpu.VMEM((128, D), jnp.float32),
        pltpu.SemaphoreType.DMA,
        pltpu.SemaphoreType.DMA,
        pltpu.VMEM_SHARED((NACC, D), jnp.float32),
    ],
)


def _sum2_body(p_hbm, h_hbm, a_v, b_v, lsem):
    c = lax.axis_index("c")
    s = lax.axis_index("s")
    wid = s * NC + c
    base = wid * (NACC // NW)

    def _chunk(q, _):
        r = base + q * 64
        pltpu.async_copy(p_hbm.at[0, pl.ds(r, 64)], a_v, lsem)
        pltpu.async_copy(p_hbm.at[1, pl.ds(r, 64)], b_v, lsem)
        pltpu.make_async_copy(p_hbm.at[0, pl.ds(0, 64)], a_v, lsem).wait()
        pltpu.make_async_copy(p_hbm.at[0, pl.ds(0, 64)], b_v, lsem).wait()

        def _row(i, _):
            for k in range(D // L):
                sl = pl.ds(k * L, L)
                a_v[i, sl] = a_v[i, sl] + b_v[i, sl]
            return 0
        lax.fori_loop(0, 64, _row, 0)
        pltpu.sync_copy(a_v, h_hbm.at[pl.ds(r, 64)])
        return 0
    lax.fori_loop(0, (NACC // NW) // 64, _chunk, 0)


_sum2 = pl.kernel(
    _sum2_body,
    out_type=jax.ShapeDtypeStruct((NACC, D), jnp.float32),
    compiler_params=_SC_PARAMS,
    mesh=_MESH,
    scratch_types=[
        pltpu.VMEM((64, D), jnp.float32),
        pltpu.VMEM((64, D), jnp.float32),
        pltpu.SemaphoreType.DMA,
    ],
)


def _final_body(f_hbm, h1_hbm, p2_hbm, o_hbm, a_v, b_v, c_v, d_v, lsem):
    c = lax.axis_index("c")
    s = lax.axis_index("s")
    wid = s * NC + c
    base = jnp.minimum(wid * 320, N - 320)

    def _chunk(q, _):
        r = base + q * 64
        pltpu.async_copy(h1_hbm.at[pl.ds(r, 64)], a_v, lsem)
        pltpu.async_copy(p2_hbm.at[0, pl.ds(r, 64)], b_v, lsem)
        pltpu.async_copy(p2_hbm.at[1, pl.ds(r, 64)], c_v, lsem)
        pltpu.async_copy(f_hbm.at[pl.ds(r, 64)], d_v, lsem)
        for _ in range(4):
            pltpu.make_async_copy(
                h1_hbm.at[pl.ds(0, 64)], a_v, lsem).wait()

        def _row(i, _):
            for k in range(D // L):
                sl = pl.ds(k * L, L)
                a_v[i, sl] = (a_v[i, sl] + b_v[i, sl]
                              + c_v[i, sl] + d_v[i, sl]) * (1.0 / 3.0)
            return 0
        lax.fori_loop(0, 64, _row, 0)
        pltpu.sync_copy(a_v, o_hbm.at[pl.ds(r, 64)])
        return 0
    lax.fori_loop(0, 5, _chunk, 0)


_final = pl.kernel(
    _final_body,
    out_type=jax.ShapeDtypeStruct((N, D), jnp.float32),
    compiler_params=_SC_PARAMS,
    mesh=_MESH,
    scratch_types=[
        pltpu.VMEM((64, D), jnp.float32),
        pltpu.VMEM((64, D), jnp.float32),
        pltpu.VMEM((64, D), jnp.float32),
        pltpu.VMEM((64, D), jnp.float32),
        pltpu.SemaphoreType.DMA,
    ],
)


def kernel(feature, edge_index, w):
    ridx = jax.random.randint(jax.random.key(1), (DROP,), 0, E)
    ridx2d = jnp.concatenate(
        [ridx, ridx[:RIDX_ROWS * 128 - DROP]]).reshape(RIDX_ROWS, 128)

    pad = jnp.arange(PADE, dtype=jnp.int32) % N
    srcp = jnp.concatenate([edge_index[0], pad]).reshape(R2D, 128)
    dstp = jnp.concatenate([edge_index[1], pad]).reshape(R2D, 128)
    w2d = w.reshape(W2DR, 128)

    pkK, wpK = _prep(srcp, dstp, w2d, ridx2d)
    p1 = _spmm(feature, pkK, wpK)
    h1 = _sum2(p1)
    p2 = _spmm(h1, pkK, wpK)
    return _final(feature, h1, p2)
